# Initial kernel scaffold; baseline (speedup 1.0000x reference)
import jax
import jax.numpy as jnp
from jax.experimental import pallas as pl


def kernel(h, src, dst, W_self_0, W_neigh_0, b_0, W_self_1, W_neigh_1, b_1):
    raise NotImplementedError("write your pallas kernel here")


# speedup vs baseline: 1.7099x; 1.7099x over previous
import functools

import jax
import jax.numpy as jnp
from jax import lax
from jax.experimental import pallas as pl
from jax.experimental.pallas import tpu as pltpu
from jax.experimental.pallas import tpu_sc as plsc

N_NODES = 10000
D = 128
NPAD = 10240
NFLAT = NPAD * NPAD
NS = 16
ZCH = 65536
STRIPE = NFLAT // NS
NZ = STRIPE // ZCH
C_ROWS = 168
FIRE = 8


def _build_adj(idx):
    mesh = plsc.VectorSubcoreMesh(
        core_axis_name="c", subcore_axis_name="s", num_cores=1
    )

    @functools.partial(
        pl.kernel,
        out_type=jax.ShapeDtypeStruct((NFLAT,), jnp.float32),
        mesh=mesh,
        scratch_types=[
            pltpu.VMEM((ZCH,), jnp.float32),
            pltpu.VMEM((C_ROWS, 128), jnp.int32),
            pltpu.VMEM((128,), jnp.float32),
            pltpu.SemaphoreType.DMA,
        ],
    )
    def scatter_kernel(idx_hbm, a_hbm, zbuf, idxv, ones, sem):
        sid = lax.axis_index("s")

        def zfill(i, _):
            zbuf[pl.ds(i * 16, 16)] = jnp.zeros((16,), jnp.float32)
            return ()

        lax.fori_loop(0, ZCH // 16, zfill, ())
        for t in range(128 // 16):
            ones[pl.ds(t * 16, 16)] = jnp.ones((16,), jnp.float32)

        base = sid * STRIPE

        def zdma(j, _):
            pltpu.sync_copy(zbuf, a_hbm.at[pl.ds(base + j * ZCH, ZCH)])
            return ()

        lax.fori_loop(0, NZ, zdma, ())

        plsc.subcore_barrier()

        pltpu.sync_copy(idx_hbm.at[sid], idxv)

        def srow(j, _):
            cps = [
                pltpu.async_copy(ones, a_hbm.at[idxv.at[j * FIRE + t]], sem)
                for t in range(FIRE)
            ]
            for cp in cps:
                cp.wait()
            return ()

        lax.fori_loop(0, C_ROWS // FIRE, srow, ())

    return scatter_kernel(idx)


BLK = 256


def _sage_layer(a, x, ws_t, wn_t, b):
    def body(a_ref, xf_ref, xb_ref, ws_ref, wn_ref, b_ref, o_ref):
        ab = a_ref[...]
        msg = jnp.dot(ab, xf_ref[...], preferred_element_type=jnp.float32)
        deg = jnp.sum(ab, axis=1, keepdims=True)
        inv = 1.0 / jnp.maximum(deg, 1.0)
        o_ref[...] = (
            jnp.dot(xb_ref[...], ws_ref[...], preferred_element_type=jnp.float32)
            + jnp.dot(msg * inv, wn_ref[...], preferred_element_type=jnp.float32)
            + b_ref[...]
        )

    return pl.pallas_call(
        body,
        grid=(NPAD // BLK,),
        in_specs=[
            pl.BlockSpec((BLK, NPAD), lambda i: (i, 0)),
            pl.BlockSpec((NPAD, D), lambda i: (0, 0)),
            pl.BlockSpec((BLK, D), lambda i: (i, 0)),
            pl.BlockSpec((D, D), lambda i: (0, 0)),
            pl.BlockSpec((D, D), lambda i: (0, 0)),
            pl.BlockSpec((1, D), lambda i: (0, 0)),
        ],
        out_specs=pl.BlockSpec((BLK, D), lambda i: (i, 0)),
        out_shape=jax.ShapeDtypeStruct((NPAD, D), jnp.float32),
    )(a, x, x, ws_t, wn_t, b)


def kernel(h, src, dst, W_self_0, W_neigh_0, b_0, W_self_1, W_neigh_1, b_1):
    n = h.shape[0]
    src = src.astype(jnp.int32)
    dst = dst.astype(jnp.int32)
    flat1 = dst * NPAD + src
    flat2 = src * NPAD + dst
    loops = jnp.arange(n, dtype=jnp.int32) * (NPAD + 1)
    idx = jnp.concatenate([flat1, flat2, loops])
    total = NS * C_ROWS * 128
    pad = jnp.full((total - idx.size,), NFLAT - 1, dtype=jnp.int32)
    idx = jnp.concatenate([idx, pad]).reshape(NS, C_ROWS, 128)

    a = _build_adj(idx).reshape(NPAD, NPAD)

    xp = jnp.pad(h, ((0, NPAD - n), (0, 0)))
    h1 = _sage_layer(a, xp, W_self_0.T, W_neigh_0.T, b_0.reshape(1, D))
    h2 = _sage_layer(a, h1, W_self_1.T, W_neigh_1.T, b_1.reshape(1, D))
    return h2[:n]


# speedup vs baseline: 1.7810x; 1.0416x over previous
import functools

import jax
import jax.numpy as jnp
from jax import lax
from jax.experimental import pallas as pl
from jax.experimental.pallas import tpu as pltpu
from jax.experimental.pallas import tpu_sc as plsc

N_NODES = 10000
D = 128
NPAD = 10240
NFLAT = NPAD * NPAD
NC = 2
NS = 16
NW = NC * NS
C_ROWS = 84
FIRE = 12


def _scatter_ones(idx, a_ref):
    mesh = plsc.VectorSubcoreMesh(core_axis_name="c", subcore_axis_name="s")

    @functools.partial(
        pl.kernel,
        out_type=(),
        mesh=mesh,
        scratch_types=[
            pltpu.VMEM((C_ROWS, 128), jnp.int32),
            pltpu.VMEM((128,), jnp.float32),
            pltpu.SemaphoreType.DMA,
        ],
    )
    def scatter_kernel(idx_hbm, a_hbm, idxv, ones, sem):
        wid = lax.axis_index("s") * NC + lax.axis_index("c")
        for t in range(128 // 16):
            ones[pl.ds(t * 16, 16)] = jnp.ones((16,), jnp.float32)
        pltpu.sync_copy(idx_hbm.at[wid], idxv)

        def srow(j, _):
            cps = [
                pltpu.async_copy(ones, a_hbm.at[idxv.at[j * FIRE + t]], sem)
                for t in range(FIRE)
            ]
            for cp in cps:
                cp.wait()
            return ()

        lax.fori_loop(0, C_ROWS // FIRE, srow, ())

    scatter_kernel(idx, a_ref)


BLK = 256


def _sage_layer(a, x, ws_t, wn_t, b):
    def body(a_ref, xf_ref, xb_ref, ws_ref, wn_ref, b_ref, o_ref):
        ab = a_ref[...]
        msg = jnp.dot(ab, xf_ref[...], preferred_element_type=jnp.float32)
        deg = jnp.sum(ab, axis=1, keepdims=True)
        inv = 1.0 / jnp.maximum(deg, 1.0)
        o_ref[...] = (
            jnp.dot(xb_ref[...], ws_ref[...], preferred_element_type=jnp.float32)
            + jnp.dot(msg * inv, wn_ref[...], preferred_element_type=jnp.float32)
            + b_ref[...]
        )

    return pl.pallas_call(
        body,
        grid=(NPAD // BLK,),
        in_specs=[
            pl.BlockSpec((BLK, NPAD), lambda i: (i, 0)),
            pl.BlockSpec((NPAD, D), lambda i: (0, 0)),
            pl.BlockSpec((BLK, D), lambda i: (i, 0)),
            pl.BlockSpec((D, D), lambda i: (0, 0)),
            pl.BlockSpec((D, D), lambda i: (0, 0)),
            pl.BlockSpec((1, D), lambda i: (0, 0)),
        ],
        out_specs=pl.BlockSpec((BLK, D), lambda i: (i, 0)),
        out_shape=jax.ShapeDtypeStruct((NPAD, D), jnp.float32),
    )(a, x, x, ws_t, wn_t, b)


def kernel(h, src, dst, W_self_0, W_neigh_0, b_0, W_self_1, W_neigh_1, b_1):
    n = h.shape[0]
    src = src.astype(jnp.int32)
    dst = dst.astype(jnp.int32)
    flat1 = dst * NPAD + src
    flat2 = src * NPAD + dst
    loops = jnp.arange(n, dtype=jnp.int32) * (NPAD + 1)
    idx = jnp.concatenate([flat1, flat2, loops])
    total = NW * C_ROWS * 128
    pad = jnp.full((total - idx.size,), NFLAT - 1, dtype=jnp.int32)
    idx = jnp.concatenate([idx, pad]).reshape(NW, C_ROWS, 128)

    a_ref = jax.new_ref(jnp.zeros((NFLAT,), jnp.float32))
    _scatter_ones(idx, a_ref)
    a = a_ref[...].reshape(NPAD, NPAD)

    xp = jnp.pad(h, ((0, NPAD - n), (0, 0)))
    h1 = _sage_layer(a, xp, W_self_0.T, W_neigh_0.T, b_0.reshape(1, D))
    h2 = _sage_layer(a, h1, W_self_1.T, W_neigh_1.T, b_1.reshape(1, D))
    return h2[:n]


# speedup vs baseline: 4.7388x; 2.6607x over previous
import functools

import jax
import jax.numpy as jnp
from jax import lax
from jax.experimental import pallas as pl
from jax.experimental.pallas import tpu as pltpu
from jax.experimental.pallas import tpu_sc as plsc

N_NODES = 10000
D = 128
NPAD = 10240
NFLAT = NPAD * NPAD
NC = 2
NS = 16
NW = NC * NS
C_ROWS = 84
FIRE = 12


def _scatter_ones(idx, a_ref):
    mesh = plsc.VectorSubcoreMesh(core_axis_name="c", subcore_axis_name="s")

    @functools.partial(
        pl.kernel,
        out_type=(),
        mesh=mesh,
        scratch_types=[
            pltpu.VMEM((C_ROWS, 128), jnp.int32),
            pltpu.VMEM((128,), jnp.float32),
            pltpu.SemaphoreType.DMA,
        ],
    )
    def scatter_kernel(idx_hbm, a_hbm, idxv, ones, sem):
        wid = lax.axis_index("s") * NC + lax.axis_index("c")
        for t in range(128 // 16):
            ones[pl.ds(t * 16, 16)] = jnp.ones((16,), jnp.float32)
        pltpu.sync_copy(idx_hbm.at[wid], idxv)

        def srow(j, _):
            cps = [
                pltpu.async_copy(ones, a_hbm.at[idxv.at[j * FIRE + t]], sem)
                for t in range(FIRE)
            ]
            for cp in cps:
                cp.wait()
            return ()

        lax.fori_loop(0, C_ROWS // FIRE, srow, ())

    scatter_kernel(idx, a_ref)


BLK = 256


def _sage_layer(a, x, ws_t, wn_t, b):
    def body(a_ref, xf_ref, xb_ref, ws_ref, wn_ref, b_ref, o_ref):
        ab = a_ref[...]
        msg = jnp.dot(ab, xf_ref[...], preferred_element_type=jnp.float32)
        deg = jnp.sum(ab, axis=1, keepdims=True)
        inv = 1.0 / jnp.maximum(deg, 1.0)
        o_ref[...] = (
            jnp.dot(xb_ref[...], ws_ref[...], preferred_element_type=jnp.float32)
            + jnp.dot(msg * inv, wn_ref[...], preferred_element_type=jnp.float32)
            + b_ref[...]
        )

    return pl.pallas_call(
        body,
        grid=(NPAD // BLK,),
        in_specs=[
            pl.BlockSpec((BLK, NPAD), lambda i: (i, 0)),
            pl.BlockSpec((NPAD, D), lambda i: (0, 0)),
            pl.BlockSpec((BLK, D), lambda i: (i, 0)),
            pl.BlockSpec((D, D), lambda i: (0, 0)),
            pl.BlockSpec((D, D), lambda i: (0, 0)),
            pl.BlockSpec((1, D), lambda i: (0, 0)),
        ],
        out_specs=pl.BlockSpec((BLK, D), lambda i: (i, 0)),
        out_shape=jax.ShapeDtypeStruct((NPAD, D), jnp.float32),
    )(a, x, x, ws_t, wn_t, b)


def kernel(h, src, dst, W_self_0, W_neigh_0, b_0, W_self_1, W_neigh_1, b_1):
    n = h.shape[0]
    src = src.astype(jnp.int32)
    dst = dst.astype(jnp.int32)
    flat1 = dst * NPAD + src
    flat2 = src * NPAD + dst
    loops = jnp.arange(n, dtype=jnp.int32) * (NPAD + 1)
    idx = jnp.concatenate([flat1, flat2, loops])
    total = NW * C_ROWS * 128
    pad = N_NODES * NPAD + jnp.arange(total - idx.size, dtype=jnp.int32)
    idx = jnp.concatenate([idx, pad]).reshape(NW, C_ROWS, 128)

    a_ref = jax.new_ref(jnp.zeros((NFLAT,), jnp.float32))
    _scatter_ones(idx, a_ref)
    a = a_ref[...].reshape(NPAD, NPAD)

    xp = jnp.pad(h, ((0, NPAD - n), (0, 0)))
    h1 = _sage_layer(a, xp, W_self_0.T, W_neigh_0.T, b_0.reshape(1, D))
    h2 = _sage_layer(a, h1, W_self_1.T, W_neigh_1.T, b_1.reshape(1, D))
    return h2[:n]
